# baseline (speedup 1.0000x reference)
import functools

import jax
import jax.numpy as jnp
from jax import lax
from jax.experimental import pallas as pl
from jax.experimental.pallas import tpu as pltpu
from jax.experimental.pallas import tpu_sc as plsc

B, H, W = 4, 512, 512
P = 50000
HW = H * W
NC, NS, L = 2, 16, 16
NW = NC * NS
TPB = NW // B
CHUNK = 6256
LAST = P - (TPB - 1) * CHUNK
PP = TPB * CHUNK
N = B * PP
GROW = 128
NROW = (CHUNK + GROW - 1) // GROW
PADN = NROW * GROW
SLICE = 2 * HW // NS
SROWS = SLICE // W


def _sc_gather(img, xa, ya, xb, yb):
    mesh = plsc.VectorSubcoreMesh(core_axis_name="c", subcore_axis_name="s")

    @functools.partial(
        pl.kernel,
        out_type=(jax.ShapeDtypeStruct((N,), jnp.float32),
                  jax.ShapeDtypeStruct((N,), jnp.float32)),
        mesh=mesh,
        scratch_types=[
            pltpu.VMEM((CHUNK,), jnp.int32),
            pltpu.VMEM((CHUNK,), jnp.int32),
            pltpu.VMEM((CHUNK,), jnp.int32),
            pltpu.VMEM((CHUNK,), jnp.int32),
            pltpu.VMEM((PADN,), jnp.int32),
            pltpu.VMEM((PADN,), jnp.int32),
            pltpu.VMEM((PADN,), jnp.float32),
            pltpu.VMEM((PADN,), jnp.float32),
            pltpu.VMEM_SHARED((2 * HW,), jnp.float32),
            pltpu.SemaphoreType.DMA,
            pltpu.SemaphoreType.DMA,
        ],
    )
    def k(img_hbm, xa_hbm, ya_hbm, xb_hbm, yb_hbm, za_hbm, zb_hbm,
          xa_v, ya_v, xb_v, yb_v, ia_v, ib_v, za_v, zb_v, smap, isem, gsem):
        c = lax.axis_index("c")
        s = lax.axis_index("s")
        wid = c * NS + s
        b = lax.shift_right_logical(wid, 3)
        t = lax.bitwise_and(wid, TPB - 1)
        base = wid * CHUNK
        coff = t * CHUNK
        boff = lax.bitwise_and(b, 1) * HW

        imgf = img_hbm.reshape(B * H, W)
        bsel = 2 * c + lax.shift_right_logical(s, 3)
        row0 = bsel * H + lax.bitwise_and(s, 7) * SROWS
        dst0 = s * SLICE

        def stage_body(r, cc):
            pltpu.async_copy(imgf.at[row0 + r, :],
                             smap.at[pl.ds(dst0 + r * W, W)], gsem)
            return cc
        lax.fori_loop(0, SROWS, stage_body, 0)

        def fetch_and_index(n):
            src = b * P + coff
            pltpu.async_copy(xa_hbm.at[pl.ds(src, n)],
                             xa_v.at[pl.ds(0, n)], isem)
            pltpu.async_copy(ya_hbm.at[pl.ds(src, n)],
                             ya_v.at[pl.ds(0, n)], isem)
            pltpu.async_copy(xb_hbm.at[pl.ds(src, n)],
                             xb_v.at[pl.ds(0, n)], isem)
            pltpu.async_copy(yb_hbm.at[pl.ds(src, n)],
                             yb_v.at[pl.ds(0, n)], isem)
            for _ in range(4):
                pltpu.make_async_copy(xa_hbm.at[pl.ds(0, n)],
                                      xa_v.at[pl.ds(0, n)], isem).wait()

            def idx_body(kk, cc):
                off = kk * L
                ia_v[pl.ds(off, L)] = (boff + ya_v[pl.ds(off, L)] * W
                                       + xa_v[pl.ds(off, L)])
                ib_v[pl.ds(off, L)] = (boff + yb_v[pl.ds(off, L)] * W
                                       + xb_v[pl.ds(off, L)])
                return cc
            lax.fori_loop(0, n // L, idx_body, 0)
            zpad = jnp.zeros((L,), jnp.int32)
            for q in range(n, PADN, L):
                ia_v[pl.ds(q, L)] = zpad
                ib_v[pl.ds(q, L)] = zpad

        @pl.when(t < TPB - 1)
        def _():
            fetch_and_index(CHUNK)

        @pl.when(t == TPB - 1)
        def _():
            fetch_and_index(LAST)

        pltpu.make_async_copy(za_hbm.at[pl.ds(0, SLICE)],
                              smap.at[pl.ds(0, SLICE)], gsem).wait()
        plsc.subcore_barrier()

        def g_body(j, cc):
            off = j * GROW
            pltpu.async_copy(smap.at[ia_v.at[pl.ds(off, GROW)]],
                             za_v.at[pl.ds(off, GROW)], gsem)
            pltpu.async_copy(smap.at[ib_v.at[pl.ds(off, GROW)]],
                             zb_v.at[pl.ds(off, GROW)], gsem)
            return cc
        lax.fori_loop(0, NROW, g_body, 0)
        pltpu.make_async_copy(smap.at[pl.ds(0, PADN)], za_v, gsem).wait()
        pltpu.make_async_copy(smap.at[pl.ds(0, PADN)], zb_v, gsem).wait()

        pltpu.async_copy(za_v.at[pl.ds(0, CHUNK)],
                         za_hbm.at[pl.ds(base, CHUNK)], isem)
        pltpu.async_copy(zb_v.at[pl.ds(0, CHUNK)],
                         zb_hbm.at[pl.ds(base, CHUNK)], isem)
        pltpu.make_async_copy(za_v.at[pl.ds(0, CHUNK)],
                              za_hbm.at[pl.ds(0, CHUNK)], isem).wait()
        pltpu.make_async_copy(za_v.at[pl.ds(0, CHUNK)],
                              zb_hbm.at[pl.ds(0, CHUNK)], isem).wait()

    return k(img, xa, ya, xb, yb)


def _loss_body(za_ref, zb_ref, o_ref, out_ref):
    acc = jnp.float32(0.0)
    for b in range(B):
        d = za_ref[pl.ds(b * PP, P)] - zb_ref[pl.ds(b * PP, P)]
        gt = o_ref[b, :].astype(jnp.float32) - 1.0
        mask = jnp.abs(gt)
        loss = mask * jnp.log(1.0 + jnp.exp(-gt * d)) + (1.0 - mask) * d * d
        acc = acc + jnp.sum(loss)
    out_ref[0] = acc / float(B * P)


def _loss(za, zb, o):
    return pl.pallas_call(
        _loss_body,
        out_shape=jax.ShapeDtypeStruct((1,), jnp.float32),
        out_specs=pl.BlockSpec(memory_space=pltpu.SMEM),
    )(za, zb, o)


def kernel(input, x_A, y_A, x_B, y_B, ordinal_relation):
    xa = x_A.astype(jnp.int32).reshape(B * P)
    ya = y_A.astype(jnp.int32).reshape(B * P)
    xb = x_B.astype(jnp.int32).reshape(B * P)
    yb = y_B.astype(jnp.int32).reshape(B * P)
    za, zb = _sc_gather(input, xa, ya, xb, yb)
    return _loss(za, zb, ordinal_relation.astype(jnp.int32))


# speedup vs baseline: 1.2346x; 1.2346x over previous
import functools

import jax
import jax.numpy as jnp
from jax import lax
from jax.experimental import pallas as pl
from jax.experimental.pallas import tpu as pltpu
from jax.experimental.pallas import tpu_sc as plsc

B, H, W = 4, 512, 512
P = 50000
HW = H * W
NC, NS, L = 2, 16, 16
NW = NC * NS
TPB = NW // B
CHUNK = 6256
PP = TPB * CHUNK
N = B * PP
GROW = 6272
NROW = (CHUNK + GROW - 1) // GROW
PADN = NROW * GROW
SLICE = 2 * HW // NS


def _prep_body(xa_ref, ya_ref, xb_ref, yb_ref, ia_ref, ib_ref):
    zpad = jnp.zeros((PP - P,), jnp.int32)
    for b in range(B):
        boff = (b % 2) * HW
        ia_ref[pl.ds(b * PP, P)] = boff + ya_ref[b, :] * W + xa_ref[b, :]
        ia_ref[pl.ds(b * PP + P, PP - P)] = zpad
        ib_ref[pl.ds(b * PP, P)] = boff + yb_ref[b, :] * W + xb_ref[b, :]
        ib_ref[pl.ds(b * PP + P, PP - P)] = zpad


def _prep(xa, ya, xb, yb):
    return pl.pallas_call(
        _prep_body,
        out_shape=(jax.ShapeDtypeStruct((N,), jnp.int32),
                   jax.ShapeDtypeStruct((N,), jnp.int32)),
    )(xa, ya, xb, yb)


def _sc_gather(img, ia, ib):
    mesh = plsc.VectorSubcoreMesh(core_axis_name="c", subcore_axis_name="s")

    @functools.partial(
        pl.kernel,
        out_type=(jax.ShapeDtypeStruct((N,), jnp.float32),
                  jax.ShapeDtypeStruct((N,), jnp.float32)),
        mesh=mesh,
        scratch_types=[
            pltpu.VMEM((PADN,), jnp.int32),
            pltpu.VMEM((PADN,), jnp.int32),
            pltpu.VMEM((PADN,), jnp.float32),
            pltpu.VMEM((PADN,), jnp.float32),
            pltpu.VMEM_SHARED((2 * HW,), jnp.float32),
            pltpu.SemaphoreType.DMA,
            pltpu.SemaphoreType.DMA,
        ],
    )
    def k(img_hbm, ia_hbm, ib_hbm, za_hbm, zb_hbm,
          ia_v, ib_v, za_v, zb_v, smap, isem, gsem):
        c = lax.axis_index("c")
        s = lax.axis_index("s")
        wid = c * NS + s
        base = wid * CHUNK

        pltpu.async_copy(ia_hbm.at[pl.ds(base, CHUNK)],
                         ia_v.at[pl.ds(0, CHUNK)], isem)
        pltpu.async_copy(ib_hbm.at[pl.ds(base, CHUNK)],
                         ib_v.at[pl.ds(0, CHUNK)], isem)
        imgf = img_hbm.reshape(B * H, W)
        bsel = 2 * c + lax.shift_right_logical(s, 3)
        row0 = bsel * H + lax.bitwise_and(s, 7) * (SLICE // W)
        dst0 = s * SLICE

        def stage_body(r, cc):
            pltpu.async_copy(imgf.at[row0 + r, :],
                             smap.at[pl.ds(dst0 + r * W, W)], gsem)
            return cc
        lax.fori_loop(0, SLICE // W, stage_body, 0)
        pltpu.make_async_copy(za_hbm.at[pl.ds(0, SLICE)],
                              smap.at[pl.ds(0, SLICE)], gsem).wait()
        pltpu.make_async_copy(ia_hbm.at[pl.ds(0, CHUNK)],
                              ia_v.at[pl.ds(0, CHUNK)], isem).wait()
        pltpu.make_async_copy(ia_hbm.at[pl.ds(0, CHUNK)],
                              ib_v.at[pl.ds(0, CHUNK)], isem).wait()
        zpad = jnp.zeros((L,), jnp.int32)
        ia_v[pl.ds(CHUNK, L)] = zpad
        ib_v[pl.ds(CHUNK, L)] = zpad
        plsc.subcore_barrier()

        def g_body(j, cc):
            off = j * GROW
            pltpu.async_copy(smap.at[ia_v.at[pl.ds(off, GROW)]],
                             za_v.at[pl.ds(off, GROW)], gsem)
            pltpu.async_copy(smap.at[ib_v.at[pl.ds(off, GROW)]],
                             zb_v.at[pl.ds(off, GROW)], gsem)
            return cc
        lax.fori_loop(0, NROW, g_body, 0)
        pltpu.make_async_copy(smap.at[pl.ds(0, PADN)], za_v, gsem).wait()
        pltpu.make_async_copy(smap.at[pl.ds(0, PADN)], zb_v, gsem).wait()

        pltpu.async_copy(za_v.at[pl.ds(0, CHUNK)],
                         za_hbm.at[pl.ds(base, CHUNK)], isem)
        pltpu.async_copy(zb_v.at[pl.ds(0, CHUNK)],
                         zb_hbm.at[pl.ds(base, CHUNK)], isem)
        pltpu.make_async_copy(za_v.at[pl.ds(0, CHUNK)],
                              za_hbm.at[pl.ds(0, CHUNK)], isem).wait()
        pltpu.make_async_copy(za_v.at[pl.ds(0, CHUNK)],
                              zb_hbm.at[pl.ds(0, CHUNK)], isem).wait()

    return k(img, ia, ib)


def _loss_body(za_ref, zb_ref, o_ref, out_ref):
    acc = jnp.float32(0.0)
    for b in range(B):
        d = za_ref[pl.ds(b * PP, P)] - zb_ref[pl.ds(b * PP, P)]
        gt = o_ref[b, :].astype(jnp.float32) - 1.0
        mask = jnp.abs(gt)
        loss = mask * jnp.log(1.0 + jnp.exp(-gt * d)) + (1.0 - mask) * d * d
        acc = acc + jnp.sum(loss)
    out_ref[0] = acc / float(B * P)


def _loss(za, zb, o):
    return pl.pallas_call(
        _loss_body,
        out_shape=jax.ShapeDtypeStruct((1,), jnp.float32),
        out_specs=pl.BlockSpec(memory_space=pltpu.SMEM),
    )(za, zb, o)


def kernel(input, x_A, y_A, x_B, y_B, ordinal_relation):
    xa = x_A.astype(jnp.int32)
    ya = y_A.astype(jnp.int32)
    xb = x_B.astype(jnp.int32)
    yb = y_B.astype(jnp.int32)
    ia, ib = _prep(xa, ya, xb, yb)
    za, zb = _sc_gather(input, ia, ib)
    return _loss(za, zb, ordinal_relation.astype(jnp.int32))


# speedup vs baseline: 1.2447x; 1.0082x over previous
import functools

import jax
import jax.numpy as jnp
from jax import lax
from jax.experimental import pallas as pl
from jax.experimental.pallas import tpu as pltpu
from jax.experimental.pallas import tpu_sc as plsc

B, H, W = 4, 512, 512
P = 50000
HW = H * W
NC, NS, L = 2, 16, 16
NW = NC * NS
TPB = NW // B
CHUNK = 6256
PP = TPB * CHUNK
N = B * PP
GROW = 6272
NROW = (CHUNK + GROW - 1) // GROW
PADN = NROW * GROW
SLICE = 2 * HW // NS


def _prep_body(xa_ref, ya_ref, xb_ref, yb_ref, ia_ref, ib_ref):
    zpad = jnp.zeros((PP - P,), jnp.int32)
    for b in range(B):
        boff = (b % 2) * HW
        ia_ref[pl.ds(b * PP, P)] = boff + ya_ref[b, :] * W + xa_ref[b, :]
        ia_ref[pl.ds(b * PP + P, PP - P)] = zpad
        ib_ref[pl.ds(b * PP, P)] = boff + yb_ref[b, :] * W + xb_ref[b, :]
        ib_ref[pl.ds(b * PP + P, PP - P)] = zpad


def _prep(xa, ya, xb, yb):
    return pl.pallas_call(
        _prep_body,
        out_shape=(jax.ShapeDtypeStruct((N,), jnp.int32),
                   jax.ShapeDtypeStruct((N,), jnp.int32)),
    )(xa, ya, xb, yb)


def _sc_gather(img, ia, ib):
    mesh = plsc.VectorSubcoreMesh(core_axis_name="c", subcore_axis_name="s")

    @functools.partial(
        pl.kernel,
        out_type=(jax.ShapeDtypeStruct((N,), jnp.float32),
                  jax.ShapeDtypeStruct((N,), jnp.float32)),
        mesh=mesh,
        scratch_types=[
            pltpu.VMEM((PADN,), jnp.int32),
            pltpu.VMEM((PADN,), jnp.int32),
            pltpu.VMEM((PADN,), jnp.float32),
            pltpu.VMEM((PADN,), jnp.float32),
            pltpu.VMEM_SHARED((2 * HW,), jnp.float32),
            pltpu.SemaphoreType.DMA,
            pltpu.SemaphoreType.DMA,
            pltpu.SemaphoreType.DMA,
        ],
    )
    def k(img_hbm, ia_hbm, ib_hbm, za_hbm, zb_hbm,
          ia_v, ib_v, za_v, zb_v, smap, isem, gsem, ssem):
        c = lax.axis_index("c")
        s = lax.axis_index("s")
        wid = c * NS + s
        base = wid * CHUNK

        pltpu.async_copy(ia_hbm.at[pl.ds(base, CHUNK)],
                         ia_v.at[pl.ds(0, CHUNK)], isem)
        pltpu.async_copy(ib_hbm.at[pl.ds(base, CHUNK)],
                         ib_v.at[pl.ds(0, CHUNK)], isem)
        imgf = img_hbm.reshape(B * H, W)
        lb = lax.bitwise_and(lax.shift_right_logical(s, 3), 1)
        hrows = H // NS

        def stage_map(m, sem):
            srow = (2 * c + m) * H + s * hrows

            def stage_body(r, cc):
                pltpu.async_copy(imgf.at[srow + r, :],
                                 smap.at[pl.ds(m * HW + (s * hrows + r) * W, W)],
                                 sem)
                return cc
            lax.fori_loop(0, hrows, stage_body, 0)

        stage_map(0, gsem)
        stage_map(1, ssem)
        pltpu.make_async_copy(ia_hbm.at[pl.ds(0, CHUNK)],
                              ia_v.at[pl.ds(0, CHUNK)], isem).wait()
        pltpu.make_async_copy(ia_hbm.at[pl.ds(0, CHUNK)],
                              ib_v.at[pl.ds(0, CHUNK)], isem).wait()
        zpad = jnp.zeros((L,), jnp.int32)
        ia_v[pl.ds(CHUNK, L)] = zpad
        ib_v[pl.ds(CHUNK, L)] = zpad
        pltpu.make_async_copy(za_hbm.at[pl.ds(0, hrows * W)],
                              smap.at[pl.ds(0, hrows * W)], gsem).wait()
        plsc.subcore_barrier()

        def fire_gathers(cc):
            pltpu.async_copy(smap.at[ia_v], za_v, gsem)
            pltpu.async_copy(smap.at[ib_v], zb_v, gsem)
            return cc

        @pl.when(lb == 0)
        def _():
            fire_gathers(0)

        pltpu.make_async_copy(za_hbm.at[pl.ds(0, hrows * W)],
                              smap.at[pl.ds(0, hrows * W)], ssem).wait()
        plsc.subcore_barrier()

        @pl.when(lb == 1)
        def _():
            fire_gathers(0)

        pltpu.make_async_copy(smap.at[pl.ds(0, PADN)], za_v, gsem).wait()
        pltpu.make_async_copy(smap.at[pl.ds(0, PADN)], zb_v, gsem).wait()

        pltpu.async_copy(za_v.at[pl.ds(0, CHUNK)],
                         za_hbm.at[pl.ds(base, CHUNK)], isem)
        pltpu.async_copy(zb_v.at[pl.ds(0, CHUNK)],
                         zb_hbm.at[pl.ds(base, CHUNK)], isem)
        pltpu.make_async_copy(za_v.at[pl.ds(0, CHUNK)],
                              za_hbm.at[pl.ds(0, CHUNK)], isem).wait()
        pltpu.make_async_copy(za_v.at[pl.ds(0, CHUNK)],
                              zb_hbm.at[pl.ds(0, CHUNK)], isem).wait()

    return k(img, ia, ib)


def _loss_body(za_ref, zb_ref, o_ref, out_ref):
    acc = jnp.float32(0.0)
    for b in range(B):
        d = za_ref[pl.ds(b * PP, P)] - zb_ref[pl.ds(b * PP, P)]
        gt = o_ref[b, :].astype(jnp.float32) - 1.0
        mask = jnp.abs(gt)
        loss = mask * jnp.log(1.0 + jnp.exp(-gt * d)) + (1.0 - mask) * d * d
        acc = acc + jnp.sum(loss)
    out_ref[0] = acc / float(B * P)


def _loss(za, zb, o):
    return pl.pallas_call(
        _loss_body,
        out_shape=jax.ShapeDtypeStruct((1,), jnp.float32),
        out_specs=pl.BlockSpec(memory_space=pltpu.SMEM),
    )(za, zb, o)


def kernel(input, x_A, y_A, x_B, y_B, ordinal_relation):
    xa = x_A.astype(jnp.int32)
    ya = y_A.astype(jnp.int32)
    xb = x_B.astype(jnp.int32)
    yb = y_B.astype(jnp.int32)
    ia, ib = _prep(xa, ya, xb, yb)
    za, zb = _sc_gather(input, ia, ib)
    return _loss(za, zb, ordinal_relation.astype(jnp.int32))
